# baseline (speedup 1.0000x reference)
import functools

import jax
import jax.numpy as jnp
from jax import lax
from jax.experimental import pallas as pl
from jax.experimental.pallas import tpu as pltpu
from jax.experimental.pallas import tpu_sc as plsc

IN_F = 1024
HID = 4096
OUT_F = 1024
E = 8
TOPK = 2
T = 2048
P = T * TOPK
BM = 256
NB = P // BM
G = NB + E - 1
BH = 1024
HB = HID // BH

NC = 2
NS = 16
NW = NC * NS


def _gating_body(x_ref, wg_ref, bg_ref, lg_ref, w_ref, i_ref):
    xv = x_ref[...]
    wg = wg_ref[...]
    logits = lax.dot_general(
        xv, wg, (((1,), (1,)), ((), ())),
        preferred_element_type=jnp.float32) + bg_ref[...]
    lg_ref[...] = logits
    ee = lax.broadcasted_iota(jnp.int32, (T, E), 1)
    m1 = jnp.max(logits, axis=1, keepdims=True)
    i1 = jnp.min(jnp.where(logits == m1, ee, E), axis=1, keepdims=True)
    masked = jnp.where(ee == i1, -jnp.inf, logits)
    m2 = jnp.max(masked, axis=1, keepdims=True)
    i2 = jnp.min(jnp.where(masked == m2, ee, E), axis=1, keepdims=True)
    e2 = jnp.exp(m2 - m1)
    s = 1.0 + e2
    w_ref[...] = jnp.concatenate([1.0 / s, e2 / s], axis=1)
    i_ref[...] = jnp.concatenate([i1, i2], axis=1)


def _gating(x_flat, Wg, bg):
    return pl.pallas_call(
        _gating_body,
        out_shape=(
            jax.ShapeDtypeStruct((T, E), jnp.float32),
            jax.ShapeDtypeStruct((T, TOPK), jnp.float32),
            jax.ShapeDtypeStruct((T, TOPK), jnp.int32),
        ),
    )(x_flat, Wg, bg.reshape(1, E))


def _route_metadata(w_tk, idx_tk):
    e_pair = idx_tk.reshape(P)
    w_pair = w_tk.reshape(P)
    oh = (e_pair[:, None] == jnp.arange(E, dtype=jnp.int32)[None, :]
          ).astype(jnp.int32)
    counts = oh.sum(axis=0)
    offsets = jnp.concatenate(
        [jnp.zeros(1, jnp.int32), jnp.cumsum(counts)]).astype(jnp.int32)
    rank_p = jnp.sum((jnp.cumsum(oh, axis=0) - 1) * oh, axis=1)
    pos = offsets[e_pair] + rank_p
    tok_pair = (jnp.arange(P, dtype=jnp.int32) // TOPK)
    tok_sorted = jnp.zeros(P, jnp.int32).at[pos].set(tok_pair)
    w_sorted = jnp.zeros(P, jnp.float32).at[pos].set(w_pair)

    first_blk = offsets[:E] // BM
    last_blk = (offsets[1:] - 1) // BM
    ntiles = jnp.where(counts > 0, last_blk - first_blk + 1, 0)
    tile_off = jnp.concatenate(
        [jnp.zeros(1, jnp.int32), jnp.cumsum(ntiles)]).astype(jnp.int32)
    t_ar = jnp.arange(G, dtype=jnp.int32)
    g_t = jnp.searchsorted(tile_off[1:], t_ar, side='right').astype(jnp.int32)
    valid = t_ar < tile_off[E]
    g_c = jnp.clip(g_t, 0, E - 1)
    b_t = jnp.where(valid, first_blk[g_c] + (t_ar - tile_off[g_c]), NB - 1)
    rs = jnp.where(valid, offsets[g_c], 0)
    re = jnp.where(valid, offsets[g_c + 1], 0)
    firsts = jnp.concatenate(
        [jnp.ones(1, jnp.int32), (b_t[1:] != b_t[:-1]).astype(jnp.int32)])
    meta = jnp.stack([g_c, b_t, firsts, rs, re])
    return meta, tok_sorted, w_sorted, pos[0::2], pos[1::2]


_GCH = 32
_GNCH = (P // NW) // _GCH


def _sc_gather_body(x_hbm, tok_hbm, a_hbm, idx_v, buf, sem):
    wid = lax.axis_index("s") * NC + lax.axis_index("c")
    base = wid * (P // NW)
    for c in range(_GNCH):
        pltpu.sync_copy(tok_hbm.at[pl.ds(base + c * _GCH, _GCH)], idx_v)
        pltpu.async_copy(x_hbm.at[idx_v], buf, sem).wait()
        pltpu.sync_copy(buf, a_hbm.at[pl.ds(base + c * _GCH, _GCH)])


def _sc_gather(x_flat, tok_sorted):
    mesh = plsc.VectorSubcoreMesh(core_axis_name="c", subcore_axis_name="s")
    f = functools.partial(
        pl.kernel,
        mesh=mesh,
        out_type=jax.ShapeDtypeStruct((P, IN_F), jnp.float32),
        scratch_types=[
            pltpu.VMEM((_GCH,), jnp.int32),
            pltpu.VMEM((_GCH, IN_F), jnp.float32),
            pltpu.SemaphoreType.DMA,
        ],
    )(_sc_gather_body)
    return f(x_flat, tok_sorted)


def _mlp_body(m_ref, a_ref, sw_ref, w1_ref, b1_ref, w2_ref, b2_ref,
              c_ref, acc_ref):
    t = pl.program_id(0)
    h = pl.program_id(1)
    wcol = jnp.concatenate([sw_ref[...]] * (IN_F // 128), axis=1)
    a_s = (a_ref[...] * wcol).astype(jnp.bfloat16)
    hb = lax.dot_general(a_s, w1_ref[0], (((1,), (1,)), ((), ())),
                         preferred_element_type=jnp.float32) + b1_ref[0]
    hb = 0.5 * hb * (1.0 + lax.erf(hb * 0.7071067811865476))
    part = lax.dot_general(hb.astype(jnp.bfloat16), w2_ref[0],
                           (((1,), (1,)), ((), ())),
                           preferred_element_type=jnp.float32)

    @pl.when(h == 0)
    def _():
        acc_ref[...] = part

    @pl.when(h > 0)
    def _():
        acc_ref[...] += part

    @pl.when(h == HB - 1)
    def _():
        r = m_ref[1, t] * BM + lax.broadcasted_iota(jnp.int32, (BM, OUT_F), 0)
        valid = (r >= m_ref[3, t]) & (r < m_ref[4, t])
        contrib = jnp.where(valid, (acc_ref[...] + b2_ref[0]) * wcol, 0.0)

        @pl.when(m_ref[2, t] == 1)
        def _():
            c_ref[...] = contrib

        @pl.when(m_ref[2, t] == 0)
        def _():
            c_ref[...] += contrib


def _grouped_mlp(meta, A, swrep, W1, b1, W2, b2):
    grid_spec = pltpu.PrefetchScalarGridSpec(
        num_scalar_prefetch=1,
        grid=(G, HB),
        in_specs=[
            pl.BlockSpec((BM, IN_F), lambda t, h, m: (m[1, t], 0)),
            pl.BlockSpec((BM, 128), lambda t, h, m: (m[1, t], 0)),
            pl.BlockSpec((1, BH, IN_F), lambda t, h, m: (m[0, t], h, 0)),
            pl.BlockSpec((1, 1, BH), lambda t, h, m: (m[0, t], 0, h)),
            pl.BlockSpec((1, OUT_F, BH), lambda t, h, m: (m[0, t], 0, h)),
            pl.BlockSpec((1, 1, OUT_F), lambda t, h, m: (m[0, t], 0, 0)),
        ],
        out_specs=pl.BlockSpec((BM, OUT_F), lambda t, h, m: (m[1, t], 0)),
        scratch_shapes=[pltpu.VMEM((BM, OUT_F), jnp.float32)],
    )
    return pl.pallas_call(
        _mlp_body,
        grid_spec=grid_spec,
        out_shape=jax.ShapeDtypeStruct((P, OUT_F), jnp.float32),
    )(meta, A, swrep, W1.astype(jnp.bfloat16), b1.reshape(E, 1, HID),
      W2.astype(jnp.bfloat16), b2.reshape(E, 1, OUT_F))


_CCH = 16
_CNCH = (T // NW) // _CCH


def _sc_combine_body(c_hbm, pa_hbm, pb_hbm, y_hbm,
                     ia_v, ib_v, bufa, bufb, sema, semb):
    wid = lax.axis_index("s") * NC + lax.axis_index("c")
    base = wid * (T // NW)
    for c in range(_CNCH):
        pltpu.sync_copy(pa_hbm.at[pl.ds(base + c * _CCH, _CCH)], ia_v)
        pltpu.sync_copy(pb_hbm.at[pl.ds(base + c * _CCH, _CCH)], ib_v)
        cpa = pltpu.async_copy(c_hbm.at[ia_v], bufa, sema)
        cpb = pltpu.async_copy(c_hbm.at[ib_v], bufb, semb)
        cpa.wait()
        cpb.wait()

        def row(i, carry):
            for j in range(OUT_F // 16):
                sl = pl.ds(j * 16, 16)
                bufa[i, sl] = bufa[i, sl] + bufb[i, sl]
            return carry

        lax.fori_loop(0, _CCH, row, 0)
        pltpu.sync_copy(bufa, y_hbm.at[pl.ds(base + c * _CCH, _CCH)])


def _sc_combine(C, posA, posB):
    mesh = plsc.VectorSubcoreMesh(core_axis_name="c", subcore_axis_name="s")
    f = functools.partial(
        pl.kernel,
        mesh=mesh,
        out_type=jax.ShapeDtypeStruct((T, OUT_F), jnp.float32),
        scratch_types=[
            pltpu.VMEM((_CCH,), jnp.int32),
            pltpu.VMEM((_CCH,), jnp.int32),
            pltpu.VMEM((_CCH, OUT_F), jnp.float32),
            pltpu.VMEM((_CCH, OUT_F), jnp.float32),
            pltpu.SemaphoreType.DMA,
            pltpu.SemaphoreType.DMA,
        ],
    )(_sc_combine_body)
    return f(C, posA, posB)


def kernel(x, Wg, bg, W1, b1, W2, b2):
    B, N, F = x.shape
    x_flat = x.reshape(T, F)
    logits, w_tk, idx_tk = _gating(x_flat, Wg, bg)
    meta, tok_sorted, w_sorted, posA, posB = _route_metadata(w_tk, idx_tk)
    A = _sc_gather(x_flat, tok_sorted)
    swrep = jnp.broadcast_to(w_sorted[:, None], (P, 128))
    C = _grouped_mlp(meta, A, swrep, W1, b1, W2, b2)
    y = _sc_combine(C, posA, posB)
    return y.reshape(B, N, OUT_F), logits


# speedup vs baseline: 1.4051x; 1.4051x over previous
import functools

import jax
import jax.numpy as jnp
from jax import lax
from jax.experimental import pallas as pl
from jax.experimental.pallas import tpu as pltpu
from jax.experimental.pallas import tpu_sc as plsc

IN_F = 1024
HID = 4096
OUT_F = 1024
E = 8
TOPK = 2
T = 2048
P = T * TOPK
BM = 512
NB = P // BM
G = NB + E - 1
BH = 1024
HB = HID // BH

NC = 2
NS = 16
NW = NC * NS


def _gating_body(x_ref, wg_ref, bg_ref, lg_ref, w_ref, i_ref):
    xv = x_ref[...]
    wg = wg_ref[...]
    logits = lax.dot_general(
        xv, wg, (((1,), (1,)), ((), ())),
        preferred_element_type=jnp.float32) + bg_ref[...]
    lg_ref[...] = logits
    ee = lax.broadcasted_iota(jnp.int32, (T, E), 1)
    m1 = jnp.max(logits, axis=1, keepdims=True)
    i1 = jnp.min(jnp.where(logits == m1, ee, E), axis=1, keepdims=True)
    masked = jnp.where(ee == i1, -jnp.inf, logits)
    m2 = jnp.max(masked, axis=1, keepdims=True)
    i2 = jnp.min(jnp.where(masked == m2, ee, E), axis=1, keepdims=True)
    e2 = jnp.exp(m2 - m1)
    s = 1.0 + e2
    w_ref[...] = jnp.concatenate([1.0 / s, e2 / s], axis=1)
    i_ref[...] = jnp.concatenate([i1, i2], axis=1)


def _gating(x_flat, Wg, bg):
    return pl.pallas_call(
        _gating_body,
        out_shape=(
            jax.ShapeDtypeStruct((T, E), jnp.float32),
            jax.ShapeDtypeStruct((T, TOPK), jnp.float32),
            jax.ShapeDtypeStruct((T, TOPK), jnp.int32),
        ),
    )(x_flat, Wg, bg.reshape(1, E))


def _route_metadata(w_tk, idx_tk):
    e_pair = idx_tk.reshape(P)
    w_pair = w_tk.reshape(P)
    oh = (e_pair[:, None] == jnp.arange(E, dtype=jnp.int32)[None, :]
          ).astype(jnp.int32)
    counts = oh.sum(axis=0)
    offsets = jnp.concatenate(
        [jnp.zeros(1, jnp.int32), jnp.cumsum(counts)]).astype(jnp.int32)
    rank_p = jnp.sum((jnp.cumsum(oh, axis=0) - 1) * oh, axis=1)
    pos = offsets[e_pair] + rank_p
    tok_pair = (jnp.arange(P, dtype=jnp.int32) // TOPK)
    tok_sorted = jnp.zeros(P, jnp.int32).at[pos].set(tok_pair)
    w_sorted = jnp.zeros(P, jnp.float32).at[pos].set(w_pair)

    first_blk = offsets[:E] // BM
    last_blk = (offsets[1:] - 1) // BM
    ntiles = jnp.where(counts > 0, last_blk - first_blk + 1, 0)
    tile_off = jnp.concatenate(
        [jnp.zeros(1, jnp.int32), jnp.cumsum(ntiles)]).astype(jnp.int32)
    t_ar = jnp.arange(G, dtype=jnp.int32)
    g_t = jnp.searchsorted(tile_off[1:], t_ar, side='right').astype(jnp.int32)
    valid = t_ar < tile_off[E]
    g_c = jnp.clip(g_t, 0, E - 1)
    b_t = jnp.where(valid, first_blk[g_c] + (t_ar - tile_off[g_c]), NB - 1)
    rs = jnp.where(valid, offsets[g_c], 0)
    re = jnp.where(valid, offsets[g_c + 1], 0)
    firsts = jnp.concatenate(
        [jnp.ones(1, jnp.int32), (b_t[1:] != b_t[:-1]).astype(jnp.int32)])
    meta = jnp.stack([g_c, b_t, firsts, rs, re])
    return meta, tok_sorted, w_sorted, pos[0::2], pos[1::2]


_GCH = 32
_GNCH = (P // NW) // _GCH


def _sc_gather_body(x_hbm, tok_hbm, a_hbm, idx_v, buf, sem):
    wid = lax.axis_index("s") * NC + lax.axis_index("c")
    base = wid * (P // NW)
    for c in range(_GNCH):
        pltpu.sync_copy(tok_hbm.at[pl.ds(base + c * _GCH, _GCH)], idx_v)
        pltpu.async_copy(x_hbm.at[idx_v], buf, sem).wait()
        pltpu.sync_copy(buf, a_hbm.at[pl.ds(base + c * _GCH, _GCH)])


def _sc_gather(x_flat, tok_sorted):
    mesh = plsc.VectorSubcoreMesh(core_axis_name="c", subcore_axis_name="s")
    f = functools.partial(
        pl.kernel,
        mesh=mesh,
        out_type=jax.ShapeDtypeStruct((P, IN_F), jnp.float32),
        scratch_types=[
            pltpu.VMEM((_GCH,), jnp.int32),
            pltpu.VMEM((_GCH, IN_F), jnp.float32),
            pltpu.SemaphoreType.DMA,
        ],
    )(_sc_gather_body)
    return f(x_flat, tok_sorted)


def _mlp_body(m_ref, a_ref, sw_ref, w1_ref, b1_ref, w2_ref, b2_ref,
              c_ref, acc_ref):
    t = pl.program_id(0)
    h = pl.program_id(1)
    wcol = jnp.concatenate([sw_ref[...]] * (IN_F // 128), axis=1)
    a_s = a_ref[...] * wcol
    hb = lax.dot_general(a_s, w1_ref[0], (((1,), (1,)), ((), ())),
                         preferred_element_type=jnp.float32) + b1_ref[0]
    hb = 0.5 * hb * (1.0 + lax.erf(hb * 0.7071067811865476))
    part = lax.dot_general(hb, w2_ref[0], (((1,), (1,)), ((), ())),
                           preferred_element_type=jnp.float32)

    @pl.when(h == 0)
    def _():
        acc_ref[...] = part

    @pl.when(h > 0)
    def _():
        acc_ref[...] += part

    @pl.when(h == HB - 1)
    def _():
        r = m_ref[1, t] * BM + lax.broadcasted_iota(jnp.int32, (BM, OUT_F), 0)
        valid = (r >= m_ref[3, t]) & (r < m_ref[4, t])
        contrib = jnp.where(valid, (acc_ref[...] + b2_ref[0]) * wcol, 0.0)

        @pl.when(m_ref[2, t] == 1)
        def _():
            c_ref[...] = contrib

        @pl.when(m_ref[2, t] == 0)
        def _():
            c_ref[...] += contrib


def _grouped_mlp(meta, A, swrep, W1, b1, W2, b2):
    grid_spec = pltpu.PrefetchScalarGridSpec(
        num_scalar_prefetch=1,
        grid=(G, HB),
        in_specs=[
            pl.BlockSpec((BM, IN_F), lambda t, h, m: (m[1, t], 0)),
            pl.BlockSpec((BM, 128), lambda t, h, m: (m[1, t], 0)),
            pl.BlockSpec((1, BH, IN_F), lambda t, h, m: (m[0, t], h, 0)),
            pl.BlockSpec((1, 1, BH), lambda t, h, m: (m[0, t], 0, h)),
            pl.BlockSpec((1, OUT_F, BH), lambda t, h, m: (m[0, t], 0, h)),
            pl.BlockSpec((1, 1, OUT_F), lambda t, h, m: (m[0, t], 0, 0)),
        ],
        out_specs=pl.BlockSpec((BM, OUT_F), lambda t, h, m: (m[1, t], 0)),
        scratch_shapes=[pltpu.VMEM((BM, OUT_F), jnp.float32)],
    )
    return pl.pallas_call(
        _mlp_body,
        grid_spec=grid_spec,
        out_shape=jax.ShapeDtypeStruct((P, OUT_F), jnp.float32),
    )(meta, A, swrep, W1, b1.reshape(E, 1, HID),
      W2, b2.reshape(E, 1, OUT_F))


_CCH = 16
_CNCH = (T // NW) // _CCH


def _sc_combine_body(c_hbm, pa_hbm, pb_hbm, y_hbm,
                     ia_v, ib_v, bufa, bufb, sema, semb):
    wid = lax.axis_index("s") * NC + lax.axis_index("c")
    base = wid * (T // NW)
    for c in range(_CNCH):
        pltpu.sync_copy(pa_hbm.at[pl.ds(base + c * _CCH, _CCH)], ia_v)
        pltpu.sync_copy(pb_hbm.at[pl.ds(base + c * _CCH, _CCH)], ib_v)
        cpa = pltpu.async_copy(c_hbm.at[ia_v], bufa, sema)
        cpb = pltpu.async_copy(c_hbm.at[ib_v], bufb, semb)
        cpa.wait()
        cpb.wait()

        def row(i, carry):
            for j in range(OUT_F // 16):
                sl = pl.ds(j * 16, 16)
                bufa[i, sl] = bufa[i, sl] + bufb[i, sl]
            return carry

        lax.fori_loop(0, _CCH, row, 0)
        pltpu.sync_copy(bufa, y_hbm.at[pl.ds(base + c * _CCH, _CCH)])


def _sc_combine(C, posA, posB):
    mesh = plsc.VectorSubcoreMesh(core_axis_name="c", subcore_axis_name="s")
    f = functools.partial(
        pl.kernel,
        mesh=mesh,
        out_type=jax.ShapeDtypeStruct((T, OUT_F), jnp.float32),
        scratch_types=[
            pltpu.VMEM((_CCH,), jnp.int32),
            pltpu.VMEM((_CCH,), jnp.int32),
            pltpu.VMEM((_CCH, OUT_F), jnp.float32),
            pltpu.VMEM((_CCH, OUT_F), jnp.float32),
            pltpu.SemaphoreType.DMA,
            pltpu.SemaphoreType.DMA,
        ],
    )(_sc_combine_body)
    return f(C, posA, posB)


def kernel(x, Wg, bg, W1, b1, W2, b2):
    B, N, F = x.shape
    x_flat = x.reshape(T, F)
    logits, w_tk, idx_tk = _gating(x_flat, Wg, bg)
    meta, tok_sorted, w_sorted, posA, posB = _route_metadata(w_tk, idx_tk)
    A = _sc_gather(x_flat, tok_sorted)
    swrep = jnp.broadcast_to(w_sorted[:, None], (P, 128))
    C = _grouped_mlp(meta, A, swrep, W1, b1, W2, b2)
    y = _sc_combine(C, posA, posB)
    return y.reshape(B, N, OUT_F), logits
